# baseline (speedup 1.0000x reference)
import functools

import jax
import jax.numpy as jnp
from jax import lax
from jax.experimental import pallas as pl
from jax.experimental.pallas import tpu as pltpu
from jax.experimental.pallas import tpu_sc as plsc

KTOP = 2
BM = 128
FT = 512
LANES = 128


def _gate_body(x_ref, wg_ref, idx_ref, w_ref):
    logits = jnp.dot(x_ref[...], wg_ref[...], preferred_element_type=jnp.float32)
    nrow = logits.shape[0]
    col = lax.broadcasted_iota(jnp.int32, (nrow, LANES), 1)
    valid = col < 8
    neg = jnp.float32(-1e30)
    masked = jnp.where(valid, logits, neg)
    m1 = jnp.max(masked, axis=1, keepdims=True)
    i1 = jnp.min(jnp.where(masked == m1, col, 999), axis=1, keepdims=True)
    masked2 = jnp.where(col == i1, neg, masked)
    m2 = jnp.max(masked2, axis=1, keepdims=True)
    i2 = jnp.min(jnp.where(masked2 == m2, col, 999), axis=1, keepdims=True)
    e2 = jnp.exp(m2 - m1)
    denom = 1.0 + e2
    w1 = 1.0 / denom
    w2 = e2 / denom
    c0 = col == 0
    c1 = col == 1
    idx_ref[...] = jnp.where(c0, i1, jnp.where(c1, i2, 0))
    w_ref[...] = jnp.where(c0, w1, jnp.where(c1, w2, 0.0))


def _gate(x, wg_pad):
    n, d = x.shape
    rb = 256
    return pl.pallas_call(
        _gate_body,
        grid=(n // rb,),
        in_specs=[
            pl.BlockSpec((rb, d), lambda i: (i, 0)),
            pl.BlockSpec((d, LANES), lambda i: (0, 0)),
        ],
        out_specs=[
            pl.BlockSpec((rb, LANES), lambda i: (i, 0)),
            pl.BlockSpec((rb, LANES), lambda i: (i, 0)),
        ],
        out_shape=[
            jax.ShapeDtypeStruct((n, LANES), jnp.int32),
            jax.ShapeDtypeStruct((n, LANES), jnp.float32),
        ],
    )(x, wg_pad)


def _sc_gather(x, row_token, g_rows):
    n, d = x.shape
    mesh = plsc.VectorSubcoreMesh(core_axis_name="c", subcore_axis_name="s")
    info = plsc.get_sparse_core_info()
    nw = info.num_cores * info.num_subcores
    per_w = g_rows // nw
    chunk = 16
    nchunks = per_w // chunk

    @functools.partial(
        pl.kernel,
        mesh=mesh,
        out_type=jax.ShapeDtypeStruct((g_rows, d), jnp.float32),
        scratch_types=[
            pltpu.VMEM((chunk,), jnp.int32),
            pltpu.VMEM((chunk, d), jnp.float32),
            pltpu.SemaphoreType.DMA,
        ],
    )
    def k(x_hbm, tok_hbm, out_hbm, idx_v, rows_v, sem):
        wid = lax.axis_index("s") * info.num_cores + lax.axis_index("c")
        base = wid * per_w

        def body(i, _):
            off = base + i * chunk
            pltpu.sync_copy(tok_hbm.at[pl.ds(off, chunk)], idx_v)
            pltpu.async_copy(x_hbm.at[idx_v], rows_v, sem).wait()
            pltpu.sync_copy(rows_v, out_hbm.at[pl.ds(off, chunk)])
            return ()

        lax.fori_loop(0, nchunks, body, ())

    return k(x, row_token)


def _sc_combine(ys, pos0, pos1):
    g_rows, d = ys.shape
    n = pos0.shape[0]
    mesh = plsc.VectorSubcoreMesh(core_axis_name="c", subcore_axis_name="s")
    info = plsc.get_sparse_core_info()
    nw = info.num_cores * info.num_subcores
    per_w = n // nw
    chunk = 16
    nchunks = per_w // chunk

    @functools.partial(
        pl.kernel,
        mesh=mesh,
        out_type=jax.ShapeDtypeStruct((n, d), jnp.float32),
        scratch_types=[
            pltpu.VMEM((chunk,), jnp.int32),
            pltpu.VMEM((chunk,), jnp.int32),
            pltpu.VMEM((chunk, d), jnp.float32),
            pltpu.VMEM((chunk, d), jnp.float32),
            pltpu.SemaphoreType.DMA,
            pltpu.SemaphoreType.DMA,
        ],
    )
    def k(ys_hbm, p0_hbm, p1_hbm, out_hbm, ia, ib, ra, rb, sa, sb):
        wid = lax.axis_index("s") * info.num_cores + lax.axis_index("c")
        base = wid * per_w

        def body(i, _):
            off = base + i * chunk
            pltpu.sync_copy(p0_hbm.at[pl.ds(off, chunk)], ia)
            pltpu.sync_copy(p1_hbm.at[pl.ds(off, chunk)], ib)
            cpa = pltpu.async_copy(ys_hbm.at[ia], ra, sa)
            cpb = pltpu.async_copy(ys_hbm.at[ib], rb, sb)
            cpa.wait()
            cpb.wait()

            def row(r, _):
                def colstep(c, _):
                    sl = pl.ds(c * 16, 16)
                    ra[r, sl] = ra[r, sl] + rb[r, sl]
                    return ()
                lax.fori_loop(0, d // 16, colstep, ())
                return ()

            lax.fori_loop(0, chunk, row, ())
            pltpu.sync_copy(ra, out_hbm.at[pl.ds(off, chunk)])
            return ()

        lax.fori_loop(0, nchunks, body, ())

    return k(ys, pos0, pos1)


def _ffn_body(be_ref, xs_ref, w1_ref, b1_ref, w2_ref, b2_ref, sc_ref,
              out_ref, h_ref, acc_ref):
    b = pl.program_id(0)
    s = pl.program_id(1)
    even = (b % 2) == 0
    nst = FT

    @pl.when(s < 8)
    def _phase1():
        j = jnp.where(even, s, 7 - s)
        xb = xs_ref[...]
        hb = jnp.dot(xb, w1_ref[0], preferred_element_type=jnp.float32)
        hb = jnp.maximum(hb + b1_ref[0], 0.0)
        h_ref[:, pl.ds(pl.multiple_of(j * nst, nst), nst)] = hb

    @pl.when(s >= 8)
    def _phase2():
        k = jnp.where(even, s - 8, 15 - s)
        hk = h_ref[:, pl.ds(pl.multiple_of(k * nst, nst), nst)]
        contrib = jnp.dot(hk, w2_ref[0], preferred_element_type=jnp.float32)

        @pl.when(s == 8)
        def _():
            acc_ref[...] = contrib

        @pl.when(s > 8)
        def _():
            acc_ref[...] = acc_ref[...] + contrib

        @pl.when(s == 15)
        def _():
            scale = sc_ref[0][:, :1]
            out_ref[...] = (acc_ref[...] + b2_ref[0]) * scale


def _ffn(xs, w1r, b1r, w2r, b2r, scale3, block_expert, nblocks):
    g_rows, d = xs.shape
    e_num, _, f = w1r.shape
    nj = f // FT

    def w1_map(b, s, be):
        j = jnp.where(b % 2 == 0,
                      jnp.where(s < 8, s, 7),
                      jnp.where(s < 8, 7 - s, 0))
        return (be[b], 0, j)

    def b1_map(b, s, be):
        j = jnp.where(b % 2 == 0,
                      jnp.where(s < 8, s, 7),
                      jnp.where(s < 8, 7 - s, 0))
        return (be[b], 0, j)

    def w2_map(b, s, be):
        k = jnp.where(b % 2 == 0,
                      jnp.where(s < 8, 0, s - 8),
                      jnp.where(s < 8, 7, 15 - s))
        return (be[b], k, 0)

    grid_spec = pltpu.PrefetchScalarGridSpec(
        num_scalar_prefetch=1,
        grid=(nblocks, 2 * nj),
        in_specs=[
            pl.BlockSpec((BM, d), lambda b, s, be: (b, 0)),
            pl.BlockSpec((1, d, FT), w1_map),
            pl.BlockSpec((1, 1, FT), b1_map),
            pl.BlockSpec((1, FT, d), w2_map),
            pl.BlockSpec((1, 1, d), lambda b, s, be: (be[b], 0, 0)),
            pl.BlockSpec((1, BM, LANES), lambda b, s, be: (b, 0, 0)),
        ],
        out_specs=pl.BlockSpec((BM, d), lambda b, s, be: (b, 0)),
        scratch_shapes=[
            pltpu.VMEM((BM, f), jnp.float32),
            pltpu.VMEM((BM, d), jnp.float32),
        ],
    )
    return pl.pallas_call(
        _ffn_body,
        grid_spec=grid_spec,
        out_shape=jax.ShapeDtypeStruct((g_rows, d), jnp.float32),
        compiler_params=pltpu.CompilerParams(
            dimension_semantics=("arbitrary", "arbitrary"),
        ),
    )(block_expert, xs, w1r, b1r, w2r, b2r, scale3)


def kernel(inputs, Wg, W1, b1, W2, b2):
    n, d = inputs.shape
    e_num = Wg.shape[1]
    f = W1.shape[2]
    p = n * KTOP

    wg_pad = jnp.pad(Wg, ((0, 0), (0, LANES - e_num)))
    top_idx, top_w = _gate(inputs, wg_pad)
    e_flat = top_idx[:, :KTOP].reshape(-1)
    w_flat = top_w[:, :KTOP].reshape(-1)

    onehot = (e_flat[:, None] == jnp.arange(e_num)[None, :]).astype(jnp.int32)
    counts = jnp.sum(onehot, axis=0)
    rank = jnp.sum((jnp.cumsum(onehot, axis=0) - onehot) * onehot, axis=1)
    pc = ((counts + BM - 1) // BM) * BM
    cum_pc = jnp.cumsum(pc)
    off = cum_pc - pc
    dest = off[e_flat] + rank

    g_rows = p + e_num * BM
    nblocks = g_rows // BM
    row_token = jnp.zeros((g_rows,), jnp.int32).at[dest].set(
        jnp.arange(p, dtype=jnp.int32) // KTOP)
    row_scale = jnp.zeros((g_rows,), jnp.float32).at[dest].set(w_flat)
    block_start = jnp.arange(nblocks, dtype=jnp.int32) * BM
    block_expert = jnp.minimum(
        jnp.sum((block_start[:, None] >= cum_pc[None, :]).astype(jnp.int32), axis=1),
        e_num - 1).astype(jnp.int32)
    pos = dest.reshape(n, KTOP)
    pos0 = pos[:, 0].astype(jnp.int32)
    pos1 = pos[:, 1].astype(jnp.int32)

    xs = _sc_gather(inputs, row_token, g_rows)

    scale3 = jnp.broadcast_to(
        row_scale.reshape(nblocks, BM, 1), (nblocks, BM, LANES))
    ys = _ffn(xs, W1, b1.reshape(e_num, 1, f), W2, b2.reshape(e_num, 1, d),
              scale3, block_expert, nblocks)

    return _sc_combine(ys, pos0, pos1)


# speedup vs baseline: 1.7036x; 1.7036x over previous
import functools

import jax
import jax.numpy as jnp
from jax import lax
from jax.experimental import pallas as pl
from jax.experimental.pallas import tpu as pltpu
from jax.experimental.pallas import tpu_sc as plsc

KTOP = 2
BM = 256
FT = 512
LANES = 128


def _gate_body(x_ref, wg_ref, idx_ref, w_ref):
    logits = jnp.dot(x_ref[...], wg_ref[...], preferred_element_type=jnp.float32)
    nrow = logits.shape[0]
    col = lax.broadcasted_iota(jnp.int32, (nrow, LANES), 1)
    valid = col < 8
    neg = jnp.float32(-1e30)
    masked = jnp.where(valid, logits, neg)
    m1 = jnp.max(masked, axis=1, keepdims=True)
    i1 = jnp.min(jnp.where(masked == m1, col, 999), axis=1, keepdims=True)
    masked2 = jnp.where(col == i1, neg, masked)
    m2 = jnp.max(masked2, axis=1, keepdims=True)
    i2 = jnp.min(jnp.where(masked2 == m2, col, 999), axis=1, keepdims=True)
    e2 = jnp.exp(m2 - m1)
    denom = 1.0 + e2
    w1 = 1.0 / denom
    w2 = e2 / denom
    c0 = col == 0
    c1 = col == 1
    idx_ref[...] = jnp.where(c0, i1, jnp.where(c1, i2, 0))
    w_ref[...] = jnp.where(c0, w1, jnp.where(c1, w2, 0.0))


def _gate(x, wg_pad):
    n, d = x.shape
    rb = 256
    return pl.pallas_call(
        _gate_body,
        grid=(n // rb,),
        in_specs=[
            pl.BlockSpec((rb, d), lambda i: (i, 0)),
            pl.BlockSpec((d, LANES), lambda i: (0, 0)),
        ],
        out_specs=[
            pl.BlockSpec((rb, LANES), lambda i: (i, 0)),
            pl.BlockSpec((rb, LANES), lambda i: (i, 0)),
        ],
        out_shape=[
            jax.ShapeDtypeStruct((n, LANES), jnp.int32),
            jax.ShapeDtypeStruct((n, LANES), jnp.float32),
        ],
    )(x, wg_pad)


def _sc_dispatch(x, dest, tok, g_rows):
    n, d = x.shape
    p_total = dest.shape[0]
    mesh = plsc.VectorSubcoreMesh(core_axis_name="c", subcore_axis_name="s")
    info = plsc.get_sparse_core_info()
    nw = info.num_cores * info.num_subcores
    per_w = p_total // nw
    chunk = 16
    nchunks = per_w // chunk

    @functools.partial(
        pl.kernel,
        mesh=mesh,
        out_type=jax.ShapeDtypeStruct((g_rows, d), jnp.float32),
        scratch_types=[
            pltpu.VMEM((chunk,), jnp.int32),
            pltpu.VMEM((chunk,), jnp.int32),
            pltpu.VMEM((chunk, d), jnp.float32),
            pltpu.SemaphoreType.DMA,
            pltpu.SemaphoreType.DMA,
        ],
    )
    def k(x_hbm, dest_hbm, tok_hbm, out_hbm, dv, tv, rows_v, sg, ss):
        wid = lax.axis_index("s") * info.num_cores + lax.axis_index("c")
        base = wid * per_w

        def body(i, _):
            off = base + i * chunk
            pltpu.sync_copy(dest_hbm.at[pl.ds(off, chunk)], dv)
            pltpu.sync_copy(tok_hbm.at[pl.ds(off, chunk)], tv)
            pltpu.async_copy(x_hbm.at[tv], rows_v, sg).wait()
            pltpu.async_copy(rows_v, out_hbm.at[dv], ss).wait()
            return ()

        lax.fori_loop(0, nchunks, body, ())

    return k(x, dest, tok)


def _sc_combine(ys, pos0, pos1, w0x, w1x):
    g_rows, d = ys.shape
    n = pos0.shape[0]
    mesh = plsc.VectorSubcoreMesh(core_axis_name="c", subcore_axis_name="s")
    info = plsc.get_sparse_core_info()
    nw = info.num_cores * info.num_subcores
    per_w = n // nw
    chunk = 16
    nchunks = per_w // chunk

    @functools.partial(
        pl.kernel,
        mesh=mesh,
        out_type=jax.ShapeDtypeStruct((n, d), jnp.float32),
        scratch_types=[
            pltpu.VMEM((chunk,), jnp.int32),
            pltpu.VMEM((chunk,), jnp.int32),
            pltpu.VMEM((chunk, 16), jnp.float32),
            pltpu.VMEM((chunk, 16), jnp.float32),
            pltpu.VMEM((chunk, d), jnp.float32),
            pltpu.VMEM((chunk, d), jnp.float32),
            pltpu.SemaphoreType.DMA,
            pltpu.SemaphoreType.DMA,
        ],
    )
    def k(ys_hbm, p0_hbm, p1_hbm, w0_hbm, w1_hbm, out_hbm,
          ia, ib, wa, wb, ra, rb, sa, sb):
        wid = lax.axis_index("s") * info.num_cores + lax.axis_index("c")
        base = wid * per_w

        def body(i, _):
            off = base + i * chunk
            pltpu.sync_copy(p0_hbm.at[pl.ds(off, chunk)], ia)
            pltpu.sync_copy(p1_hbm.at[pl.ds(off, chunk)], ib)
            pltpu.sync_copy(w0_hbm.at[pl.ds(off, chunk)], wa)
            pltpu.sync_copy(w1_hbm.at[pl.ds(off, chunk)], wb)
            cpa = pltpu.async_copy(ys_hbm.at[ia], ra, sa)
            cpb = pltpu.async_copy(ys_hbm.at[ib], rb, sb)
            cpa.wait()
            cpb.wait()

            def row(r, _):
                bwa = wa[r]
                bwb = wb[r]

                def colstep(c, _):
                    sl = pl.ds(c * 16, 16)
                    ra[r, sl] = ra[r, sl] * bwa + rb[r, sl] * bwb
                    return ()

                lax.fori_loop(0, d // 16, colstep, ())
                return ()

            lax.fori_loop(0, chunk, row, ())
            pltpu.sync_copy(ra, out_hbm.at[pl.ds(off, chunk)])
            return ()

        lax.fori_loop(0, nchunks, body, ())

    return k(ys, pos0, pos1, w0x, w1x)


def _ffn_body(be_ref, xs_ref, w1_ref, b1_ref, w2_ref, b2_ref,
              out_ref, acc_ref):
    s = pl.program_id(1)
    xb = xs_ref[...].astype(jnp.bfloat16)
    hb = jnp.dot(xb, w1_ref[0].astype(jnp.bfloat16),
                 preferred_element_type=jnp.float32)
    hb = jnp.maximum(hb + b1_ref[0], 0.0)
    contrib = jnp.dot(hb.astype(jnp.bfloat16),
                      w2_ref[0].astype(jnp.bfloat16),
                      preferred_element_type=jnp.float32)

    @pl.when(s == 0)
    def _():
        acc_ref[...] = contrib

    @pl.when(s > 0)
    def _():
        acc_ref[...] = acc_ref[...] + contrib

    @pl.when(s == 7)
    def _():
        out_ref[...] = acc_ref[...] + b2_ref[0]


def _ffn(xs, w1r, b1r, w2r, b2r, block_expert, nblocks):
    g_rows, d = xs.shape
    e_num, _, f = w1r.shape
    nj = f // FT

    def jmap(b, s):
        return jnp.where(b % 2 == 0, s, (nj - 1) - s)

    grid_spec = pltpu.PrefetchScalarGridSpec(
        num_scalar_prefetch=1,
        grid=(nblocks, nj),
        in_specs=[
            pl.BlockSpec((BM, d), lambda b, s, be: (b, 0)),
            pl.BlockSpec((1, d, FT), lambda b, s, be: (be[b], 0, jmap(b, s))),
            pl.BlockSpec((1, 1, FT), lambda b, s, be: (be[b], 0, jmap(b, s))),
            pl.BlockSpec((1, FT, d), lambda b, s, be: (be[b], jmap(b, s), 0)),
            pl.BlockSpec((1, 1, d), lambda b, s, be: (be[b], 0, 0)),
        ],
        out_specs=pl.BlockSpec((BM, d), lambda b, s, be: (b, 0)),
        scratch_shapes=[
            pltpu.VMEM((BM, d), jnp.float32),
        ],
    )
    return pl.pallas_call(
        _ffn_body,
        grid_spec=grid_spec,
        out_shape=jax.ShapeDtypeStruct((g_rows, d), jnp.float32),
        compiler_params=pltpu.CompilerParams(
            dimension_semantics=("arbitrary", "arbitrary"),
        ),
    )(block_expert, xs, w1r, b1r, w2r, b2r)


def kernel(inputs, Wg, W1, b1, W2, b2):
    n, d = inputs.shape
    e_num = Wg.shape[1]
    f = W1.shape[2]
    p = n * KTOP

    wg_pad = jnp.pad(Wg, ((0, 0), (0, LANES - e_num)))
    top_idx, top_w = _gate(inputs, wg_pad)
    e_flat = top_idx[:, :KTOP].reshape(-1)

    onehot = (e_flat[:, None] == jnp.arange(e_num)[None, :]).astype(jnp.int32)
    counts = jnp.sum(onehot, axis=0)
    rank = jnp.sum((jnp.cumsum(onehot, axis=0) - onehot) * onehot, axis=1)
    pc = ((counts + BM - 1) // BM) * BM
    cum_pc = jnp.cumsum(pc)
    off = cum_pc - pc
    dest = (off[e_flat] + rank).astype(jnp.int32)

    g_rows = p + e_num * BM
    nblocks = g_rows // BM
    block_start = jnp.arange(nblocks, dtype=jnp.int32) * BM
    block_expert = jnp.minimum(
        jnp.sum((block_start[:, None] >= cum_pc[None, :]).astype(jnp.int32), axis=1),
        e_num - 1).astype(jnp.int32)
    pos = dest.reshape(n, KTOP)

    tok = (jnp.arange(p, dtype=jnp.int32) // KTOP).astype(jnp.int32)
    xs = _sc_dispatch(inputs, dest, tok, g_rows)

    ys = _ffn(xs, W1, b1.reshape(e_num, 1, f), W2, b2.reshape(e_num, 1, d),
              block_expert, nblocks)

    w0x = jnp.broadcast_to(top_w[:, 0:1], (n, 16))
    w1x = jnp.broadcast_to(top_w[:, 1:2], (n, 16))
    return _sc_combine(ys, pos[:, 0], pos[:, 1], w0x, w1x)


# speedup vs baseline: 1.9652x; 1.1536x over previous
import functools

import jax
import jax.numpy as jnp
from jax import lax
from jax.experimental import pallas as pl
from jax.experimental.pallas import tpu as pltpu
from jax.experimental.pallas import tpu_sc as plsc

KTOP = 2
BM = 512
FT = 512
LANES = 128


def _gate_body(x_ref, wg_ref, idx_ref, w_ref):
    logits = jnp.dot(x_ref[...], wg_ref[...], preferred_element_type=jnp.float32)
    nrow = logits.shape[0]
    col = lax.broadcasted_iota(jnp.int32, (nrow, LANES), 1)
    valid = col < 8
    neg = jnp.float32(-1e30)
    masked = jnp.where(valid, logits, neg)
    m1 = jnp.max(masked, axis=1, keepdims=True)
    i1 = jnp.min(jnp.where(masked == m1, col, 999), axis=1, keepdims=True)
    masked2 = jnp.where(col == i1, neg, masked)
    m2 = jnp.max(masked2, axis=1, keepdims=True)
    i2 = jnp.min(jnp.where(masked2 == m2, col, 999), axis=1, keepdims=True)
    e2 = jnp.exp(m2 - m1)
    denom = 1.0 + e2
    w1 = 1.0 / denom
    w2 = e2 / denom
    c0 = col == 0
    c1 = col == 1
    idx_ref[...] = jnp.where(c0, i1, jnp.where(c1, i2, 0))
    w_ref[...] = jnp.where(c0, w1, jnp.where(c1, w2, 0.0))


def _gate(x, wg_pad):
    n, d = x.shape
    rb = 256
    return pl.pallas_call(
        _gate_body,
        grid=(n // rb,),
        in_specs=[
            pl.BlockSpec((rb, d), lambda i: (i, 0)),
            pl.BlockSpec((d, LANES), lambda i: (0, 0)),
        ],
        out_specs=[
            pl.BlockSpec((rb, LANES), lambda i: (i, 0)),
            pl.BlockSpec((rb, LANES), lambda i: (i, 0)),
        ],
        out_shape=[
            jax.ShapeDtypeStruct((n, LANES), jnp.int32),
            jax.ShapeDtypeStruct((n, LANES), jnp.float32),
        ],
    )(x, wg_pad)


def _sc_dispatch(x, dest, tok, g_rows):
    n, d = x.shape
    p_total = dest.shape[0]
    mesh = plsc.VectorSubcoreMesh(core_axis_name="c", subcore_axis_name="s")
    info = plsc.get_sparse_core_info()
    nw = info.num_cores * info.num_subcores
    per_w = p_total // nw
    chunk = 16
    nchunks = per_w // chunk

    @functools.partial(
        pl.kernel,
        mesh=mesh,
        out_type=jax.ShapeDtypeStruct((g_rows, d), jnp.float32),
        scratch_types=[
            pltpu.VMEM((chunk,), jnp.int32),
            pltpu.VMEM((chunk,), jnp.int32),
            pltpu.VMEM((chunk, d), jnp.float32),
            pltpu.SemaphoreType.DMA,
            pltpu.SemaphoreType.DMA,
        ],
    )
    def k(x_hbm, dest_hbm, tok_hbm, out_hbm, dv, tv, rows_v, sg, ss):
        wid = lax.axis_index("s") * info.num_cores + lax.axis_index("c")
        base = wid * per_w

        def body(i, _):
            off = base + i * chunk
            pltpu.sync_copy(dest_hbm.at[pl.ds(off, chunk)], dv)
            pltpu.sync_copy(tok_hbm.at[pl.ds(off, chunk)], tv)
            pltpu.async_copy(x_hbm.at[tv], rows_v, sg).wait()
            pltpu.async_copy(rows_v, out_hbm.at[dv], ss).wait()
            return ()

        lax.fori_loop(0, nchunks, body, ())

    return k(x, dest, tok)


def _sc_combine(ys, pos0, pos1, w0x, w1x):
    g_rows, d = ys.shape
    n = pos0.shape[0]
    mesh = plsc.VectorSubcoreMesh(core_axis_name="c", subcore_axis_name="s")
    info = plsc.get_sparse_core_info()
    nw = info.num_cores * info.num_subcores
    per_w = n // nw
    chunk = 16
    nchunks = per_w // chunk

    @functools.partial(
        pl.kernel,
        mesh=mesh,
        out_type=jax.ShapeDtypeStruct((n, d), jnp.float32),
        scratch_types=[
            pltpu.VMEM((chunk,), jnp.int32),
            pltpu.VMEM((chunk,), jnp.int32),
            pltpu.VMEM((chunk, 16), jnp.float32),
            pltpu.VMEM((chunk, 16), jnp.float32),
            pltpu.VMEM((chunk, d), jnp.float32),
            pltpu.VMEM((chunk, d), jnp.float32),
            pltpu.SemaphoreType.DMA,
            pltpu.SemaphoreType.DMA,
        ],
    )
    def k(ys_hbm, p0_hbm, p1_hbm, w0_hbm, w1_hbm, out_hbm,
          ia, ib, wa, wb, ra, rb, sa, sb):
        wid = lax.axis_index("s") * info.num_cores + lax.axis_index("c")
        base = wid * per_w

        def body(i, _):
            off = base + i * chunk
            pltpu.sync_copy(p0_hbm.at[pl.ds(off, chunk)], ia)
            pltpu.sync_copy(p1_hbm.at[pl.ds(off, chunk)], ib)
            pltpu.sync_copy(w0_hbm.at[pl.ds(off, chunk)], wa)
            pltpu.sync_copy(w1_hbm.at[pl.ds(off, chunk)], wb)
            cpa = pltpu.async_copy(ys_hbm.at[ia], ra, sa)
            cpb = pltpu.async_copy(ys_hbm.at[ib], rb, sb)
            cpa.wait()
            cpb.wait()

            def row(r, _):
                bwa = wa[r]
                bwb = wb[r]

                def colstep(c, _):
                    sl = pl.ds(c * 16, 16)
                    ra[r, sl] = ra[r, sl] * bwa + rb[r, sl] * bwb
                    return ()

                lax.fori_loop(0, d // 16, colstep, ())
                return ()

            lax.fori_loop(0, chunk, row, ())
            pltpu.sync_copy(ra, out_hbm.at[pl.ds(off, chunk)])
            return ()

        lax.fori_loop(0, nchunks, body, ())

    return k(ys, pos0, pos1, w0x, w1x)


def _ffn_body(be_ref, xs_ref, w1_ref, b1_ref, w2_ref, b2_ref,
              out_ref, acc_ref):
    s = pl.program_id(1)
    xb = xs_ref[...].astype(jnp.bfloat16)
    hb = jnp.dot(xb, w1_ref[0].astype(jnp.bfloat16),
                 preferred_element_type=jnp.float32)
    hb = jnp.maximum(hb + b1_ref[0], 0.0)
    contrib = jnp.dot(hb.astype(jnp.bfloat16),
                      w2_ref[0].astype(jnp.bfloat16),
                      preferred_element_type=jnp.float32)

    @pl.when(s == 0)
    def _():
        acc_ref[...] = contrib

    @pl.when(s > 0)
    def _():
        acc_ref[...] = acc_ref[...] + contrib

    @pl.when(s == 7)
    def _():
        out_ref[...] = acc_ref[...] + b2_ref[0]


def _ffn(xs, w1r, b1r, w2r, b2r, block_expert, nblocks):
    g_rows, d = xs.shape
    e_num, _, f = w1r.shape
    nj = f // FT

    def jmap(b, s):
        return jnp.where(b % 2 == 0, s, (nj - 1) - s)

    grid_spec = pltpu.PrefetchScalarGridSpec(
        num_scalar_prefetch=1,
        grid=(nblocks, nj),
        in_specs=[
            pl.BlockSpec((BM, d), lambda b, s, be: (b, 0)),
            pl.BlockSpec((1, d, FT), lambda b, s, be: (be[b], 0, jmap(b, s))),
            pl.BlockSpec((1, 1, FT), lambda b, s, be: (be[b], 0, jmap(b, s))),
            pl.BlockSpec((1, FT, d), lambda b, s, be: (be[b], jmap(b, s), 0)),
            pl.BlockSpec((1, 1, d), lambda b, s, be: (be[b], 0, 0)),
        ],
        out_specs=pl.BlockSpec((BM, d), lambda b, s, be: (b, 0)),
        scratch_shapes=[
            pltpu.VMEM((BM, d), jnp.float32),
        ],
    )
    return pl.pallas_call(
        _ffn_body,
        grid_spec=grid_spec,
        out_shape=jax.ShapeDtypeStruct((g_rows, d), jnp.float32),
        compiler_params=pltpu.CompilerParams(
            dimension_semantics=("arbitrary", "arbitrary"),
        ),
    )(block_expert, xs, w1r, b1r, w2r, b2r)


def kernel(inputs, Wg, W1, b1, W2, b2):
    n, d = inputs.shape
    e_num = Wg.shape[1]
    f = W1.shape[2]
    p = n * KTOP

    wg_pad = jnp.pad(Wg, ((0, 0), (0, LANES - e_num)))
    top_idx, top_w = _gate(inputs, wg_pad)
    e_flat = top_idx[:, :KTOP].reshape(-1)

    onehot = (e_flat[:, None] == jnp.arange(e_num)[None, :]).astype(jnp.int32)
    counts = jnp.sum(onehot, axis=0)
    rank = jnp.sum((jnp.cumsum(onehot, axis=0) - onehot) * onehot, axis=1)
    pc = ((counts + BM - 1) // BM) * BM
    cum_pc = jnp.cumsum(pc)
    off = cum_pc - pc
    dest = (off[e_flat] + rank).astype(jnp.int32)

    g_rows = p + e_num * BM
    nblocks = g_rows // BM
    block_start = jnp.arange(nblocks, dtype=jnp.int32) * BM
    block_expert = jnp.minimum(
        jnp.sum((block_start[:, None] >= cum_pc[None, :]).astype(jnp.int32), axis=1),
        e_num - 1).astype(jnp.int32)
    pos = dest.reshape(n, KTOP)

    tok = (jnp.arange(p, dtype=jnp.int32) // KTOP).astype(jnp.int32)
    xs = _sc_dispatch(inputs, dest, tok, g_rows)

    ys = _ffn(xs, W1, b1.reshape(e_num, 1, f), W2, b2.reshape(e_num, 1, d),
              block_expert, nblocks)

    w0x = jnp.broadcast_to(top_w[:, 0:1], (n, 16))
    w1x = jnp.broadcast_to(top_w[:, 1:2], (n, 16))
    return _sc_combine(ys, pos[:, 0], pos[:, 1], w0x, w1x)


# speedup vs baseline: 2.1529x; 1.0955x over previous
import functools

import jax
import jax.numpy as jnp
from jax import lax
from jax.experimental import pallas as pl
from jax.experimental.pallas import tpu as pltpu
from jax.experimental.pallas import tpu_sc as plsc

KTOP = 2
BM = 512
FT = 1024
LANES = 128


def _gate_body(x_ref, wg_ref, idx_ref, w_ref):
    logits = jnp.dot(x_ref[...], wg_ref[...], preferred_element_type=jnp.float32)
    nrow = logits.shape[0]
    col = lax.broadcasted_iota(jnp.int32, (nrow, LANES), 1)
    valid = col < 8
    neg = jnp.float32(-1e30)
    masked = jnp.where(valid, logits, neg)
    m1 = jnp.max(masked, axis=1, keepdims=True)
    i1 = jnp.min(jnp.where(masked == m1, col, 999), axis=1, keepdims=True)
    masked2 = jnp.where(col == i1, neg, masked)
    m2 = jnp.max(masked2, axis=1, keepdims=True)
    i2 = jnp.min(jnp.where(masked2 == m2, col, 999), axis=1, keepdims=True)
    e2 = jnp.exp(m2 - m1)
    denom = 1.0 + e2
    w1 = 1.0 / denom
    w2 = e2 / denom
    c0 = col == 0
    c1 = col == 1
    idx_ref[...] = jnp.where(c0, i1, jnp.where(c1, i2, 0))
    w_ref[...] = jnp.where(c0, w1, jnp.where(c1, w2, 0.0))


def _gate(x, wg_pad):
    n, d = x.shape
    rb = 256
    return pl.pallas_call(
        _gate_body,
        grid=(n // rb,),
        in_specs=[
            pl.BlockSpec((rb, d), lambda i: (i, 0)),
            pl.BlockSpec((d, LANES), lambda i: (0, 0)),
        ],
        out_specs=[
            pl.BlockSpec((rb, LANES), lambda i: (i, 0)),
            pl.BlockSpec((rb, LANES), lambda i: (i, 0)),
        ],
        out_shape=[
            jax.ShapeDtypeStruct((n, LANES), jnp.int32),
            jax.ShapeDtypeStruct((n, LANES), jnp.float32),
        ],
    )(x, wg_pad)


def _sc_dispatch(x, dest, tok, g_rows):
    n, d = x.shape
    p_total = dest.shape[0]
    mesh = plsc.VectorSubcoreMesh(core_axis_name="c", subcore_axis_name="s")
    info = plsc.get_sparse_core_info()
    nw = info.num_cores * info.num_subcores
    per_w = p_total // nw
    chunk = 16
    nchunks = per_w // chunk

    @functools.partial(
        pl.kernel,
        mesh=mesh,
        out_type=jax.ShapeDtypeStruct((g_rows, d), jnp.float32),
        scratch_types=[
            pltpu.VMEM((chunk,), jnp.int32),
            pltpu.VMEM((chunk,), jnp.int32),
            pltpu.VMEM((chunk, d), jnp.float32),
            pltpu.SemaphoreType.DMA,
            pltpu.SemaphoreType.DMA,
        ],
    )
    def k(x_hbm, dest_hbm, tok_hbm, out_hbm, dv, tv, rows_v, sg, ss):
        wid = lax.axis_index("s") * info.num_cores + lax.axis_index("c")
        base = wid * per_w

        def body(i, _):
            off = base + i * chunk
            pltpu.sync_copy(dest_hbm.at[pl.ds(off, chunk)], dv)
            pltpu.sync_copy(tok_hbm.at[pl.ds(off, chunk)], tv)
            pltpu.async_copy(x_hbm.at[tv], rows_v, sg).wait()
            pltpu.async_copy(rows_v, out_hbm.at[dv], ss).wait()
            return ()

        lax.fori_loop(0, nchunks, body, ())

    return k(x, dest, tok)


def _sc_combine(ys, pos0, pos1, w0x, w1x):
    g_rows, d = ys.shape
    n = pos0.shape[0]
    mesh = plsc.VectorSubcoreMesh(core_axis_name="c", subcore_axis_name="s")
    info = plsc.get_sparse_core_info()
    nw = info.num_cores * info.num_subcores
    per_w = n // nw
    chunk = 16
    nchunks = per_w // chunk

    @functools.partial(
        pl.kernel,
        mesh=mesh,
        out_type=jax.ShapeDtypeStruct((n, d), jnp.float32),
        scratch_types=[
            pltpu.VMEM((chunk,), jnp.int32),
            pltpu.VMEM((chunk,), jnp.int32),
            pltpu.VMEM((chunk, 16), jnp.float32),
            pltpu.VMEM((chunk, 16), jnp.float32),
            pltpu.VMEM((chunk, d), jnp.float32),
            pltpu.VMEM((chunk, d), jnp.float32),
            pltpu.SemaphoreType.DMA,
            pltpu.SemaphoreType.DMA,
        ],
    )
    def k(ys_hbm, p0_hbm, p1_hbm, w0_hbm, w1_hbm, out_hbm,
          ia, ib, wa, wb, ra, rb, sa, sb):
        wid = lax.axis_index("s") * info.num_cores + lax.axis_index("c")
        base = wid * per_w

        def body(i, _):
            off = base + i * chunk
            pltpu.sync_copy(p0_hbm.at[pl.ds(off, chunk)], ia)
            pltpu.sync_copy(p1_hbm.at[pl.ds(off, chunk)], ib)
            pltpu.sync_copy(w0_hbm.at[pl.ds(off, chunk)], wa)
            pltpu.sync_copy(w1_hbm.at[pl.ds(off, chunk)], wb)
            cpa = pltpu.async_copy(ys_hbm.at[ia], ra, sa)
            cpb = pltpu.async_copy(ys_hbm.at[ib], rb, sb)
            cpa.wait()
            cpb.wait()

            def row(r, _):
                bwa = wa[r]
                bwb = wb[r]

                def colstep(c, _):
                    sl = pl.ds(c * 16, 16)
                    ra[r, sl] = ra[r, sl] * bwa + rb[r, sl] * bwb
                    return ()

                lax.fori_loop(0, d // 16, colstep, ())
                return ()

            lax.fori_loop(0, chunk, row, ())
            pltpu.sync_copy(ra, out_hbm.at[pl.ds(off, chunk)])
            return ()

        lax.fori_loop(0, nchunks, body, ())

    return k(ys, pos0, pos1, w0x, w1x)


def _ffn_body(be_ref, xs_ref, w1_ref, b1_ref, w2_ref, b2_ref,
              out_ref, acc_ref):
    s = pl.program_id(1)
    xb = xs_ref[...].astype(jnp.bfloat16)
    hb = jnp.dot(xb, w1_ref[0].astype(jnp.bfloat16),
                 preferred_element_type=jnp.float32)
    hb = jnp.maximum(hb + b1_ref[0], 0.0)
    contrib = jnp.dot(hb.astype(jnp.bfloat16),
                      w2_ref[0].astype(jnp.bfloat16),
                      preferred_element_type=jnp.float32)

    @pl.when(s == 0)
    def _():
        acc_ref[...] = contrib

    @pl.when(s > 0)
    def _():
        acc_ref[...] = acc_ref[...] + contrib

    @pl.when(s == pl.num_programs(1) - 1)
    def _():
        out_ref[...] = acc_ref[...] + b2_ref[0]


def _ffn(xs, w1r, b1r, w2r, b2r, block_expert, nblocks):
    g_rows, d = xs.shape
    e_num, _, f = w1r.shape
    nj = f // FT

    def jmap(b, s):
        return jnp.where(b % 2 == 0, s, (nj - 1) - s)

    grid_spec = pltpu.PrefetchScalarGridSpec(
        num_scalar_prefetch=1,
        grid=(nblocks, nj),
        in_specs=[
            pl.BlockSpec((BM, d), lambda b, s, be: (b, 0)),
            pl.BlockSpec((1, d, FT), lambda b, s, be: (be[b], 0, jmap(b, s))),
            pl.BlockSpec((1, 1, FT), lambda b, s, be: (be[b], 0, jmap(b, s))),
            pl.BlockSpec((1, FT, d), lambda b, s, be: (be[b], jmap(b, s), 0)),
            pl.BlockSpec((1, 1, d), lambda b, s, be: (be[b], 0, 0)),
        ],
        out_specs=pl.BlockSpec((BM, d), lambda b, s, be: (b, 0)),
        scratch_shapes=[
            pltpu.VMEM((BM, d), jnp.float32),
        ],
    )
    return pl.pallas_call(
        _ffn_body,
        grid_spec=grid_spec,
        out_shape=jax.ShapeDtypeStruct((g_rows, d), jnp.float32),
        compiler_params=pltpu.CompilerParams(
            dimension_semantics=("arbitrary", "arbitrary"),
        ),
    )(block_expert, xs, w1r, b1r, w2r, b2r)


def kernel(inputs, Wg, W1, b1, W2, b2):
    n, d = inputs.shape
    e_num = Wg.shape[1]
    f = W1.shape[2]
    p = n * KTOP

    wg_pad = jnp.pad(Wg, ((0, 0), (0, LANES - e_num)))
    top_idx, top_w = _gate(inputs, wg_pad)
    e_flat = top_idx[:, :KTOP].reshape(-1)

    onehot = (e_flat[:, None] == jnp.arange(e_num)[None, :]).astype(jnp.int32)
    counts = jnp.sum(onehot, axis=0)
    rank = jnp.sum((jnp.cumsum(onehot, axis=0) - onehot) * onehot, axis=1)
    pc = ((counts + BM - 1) // BM) * BM
    cum_pc = jnp.cumsum(pc)
    off = cum_pc - pc
    dest = (off[e_flat] + rank).astype(jnp.int32)

    g_rows = p + e_num * BM
    nblocks = g_rows // BM
    block_start = jnp.arange(nblocks, dtype=jnp.int32) * BM
    block_expert = jnp.minimum(
        jnp.sum((block_start[:, None] >= cum_pc[None, :]).astype(jnp.int32), axis=1),
        e_num - 1).astype(jnp.int32)
    pos = dest.reshape(n, KTOP)

    tok = (jnp.arange(p, dtype=jnp.int32) // KTOP).astype(jnp.int32)
    xs = _sc_dispatch(inputs, dest, tok, g_rows)

    ys = _ffn(xs, W1, b1.reshape(e_num, 1, f), W2, b2.reshape(e_num, 1, d),
              block_expert, nblocks)

    w0x = jnp.broadcast_to(top_w[:, 0:1], (n, 16))
    w1x = jnp.broadcast_to(top_w[:, 1:2], (n, 16))
    return _sc_combine(ys, pos[:, 0], pos[:, 1], w0x, w1x)


# speedup vs baseline: 2.3570x; 1.0948x over previous
import functools

import jax
import jax.numpy as jnp
from jax import lax
from jax.experimental import pallas as pl
from jax.experimental.pallas import tpu as pltpu
from jax.experimental.pallas import tpu_sc as plsc

KTOP = 2
BM = 512
FT = 1024
LANES = 128


def _gate_body(x_ref, wg_ref, idx_ref, w_ref):
    logits = jnp.dot(x_ref[...], wg_ref[...], preferred_element_type=jnp.float32)
    nrow = logits.shape[0]
    col = lax.broadcasted_iota(jnp.int32, (nrow, LANES), 1)
    valid = col < 8
    neg = jnp.float32(-1e30)
    masked = jnp.where(valid, logits, neg)
    m1 = jnp.max(masked, axis=1, keepdims=True)
    i1 = jnp.min(jnp.where(masked == m1, col, 999), axis=1, keepdims=True)
    masked2 = jnp.where(col == i1, neg, masked)
    m2 = jnp.max(masked2, axis=1, keepdims=True)
    i2 = jnp.min(jnp.where(masked2 == m2, col, 999), axis=1, keepdims=True)
    e2 = jnp.exp(m2 - m1)
    denom = 1.0 + e2
    w1 = 1.0 / denom
    w2 = e2 / denom
    c0 = col == 0
    c1 = col == 1
    idx_ref[...] = jnp.where(c0, i1, jnp.where(c1, i2, 0))
    w_ref[...] = jnp.where(c0, w1, jnp.where(c1, w2, 0.0))


def _gate(x, wg_pad):
    n, d = x.shape
    rb = 256
    return pl.pallas_call(
        _gate_body,
        grid=(n // rb,),
        in_specs=[
            pl.BlockSpec((rb, d), lambda i: (i, 0)),
            pl.BlockSpec((d, LANES), lambda i: (0, 0)),
        ],
        out_specs=[
            pl.BlockSpec((rb, LANES), lambda i: (i, 0)),
            pl.BlockSpec((rb, LANES), lambda i: (i, 0)),
        ],
        out_shape=[
            jax.ShapeDtypeStruct((n, LANES), jnp.int32),
            jax.ShapeDtypeStruct((n, LANES), jnp.float32),
        ],
    )(x, wg_pad)


def _sc_dispatch(x, dest, tok, g_rows):
    n, d = x.shape
    p_total = dest.shape[0]
    mesh = plsc.VectorSubcoreMesh(core_axis_name="c", subcore_axis_name="s")
    info = plsc.get_sparse_core_info()
    nw = info.num_cores * info.num_subcores
    per_w = p_total // nw
    chunk = 16
    nchunks = per_w // chunk

    @functools.partial(
        pl.kernel,
        mesh=mesh,
        out_type=jax.ShapeDtypeStruct((g_rows, d), jnp.float32),
        scratch_types=[
            pltpu.VMEM((2, chunk), jnp.int32),
            pltpu.VMEM((2, chunk), jnp.int32),
            pltpu.VMEM((2, chunk, d), jnp.float32),
            pltpu.SemaphoreType.DMA,
            pltpu.SemaphoreType.DMA,
            pltpu.SemaphoreType.DMA,
            pltpu.SemaphoreType.DMA,
        ],
    )
    def k(x_hbm, dest_hbm, tok_hbm, out_hbm, dv, tv, rows_v, sg0, sg1, ss0, ss1):
        wid = lax.axis_index("s") * info.num_cores + lax.axis_index("c")
        base = wid * per_w
        sgs = (sg0, sg1)
        sss = (ss0, ss1)

        def issue(i, sl):
            off = base + i * chunk
            pltpu.sync_copy(dest_hbm.at[pl.ds(off, chunk)], dv.at[sl])
            pltpu.sync_copy(tok_hbm.at[pl.ds(off, chunk)], tv.at[sl])
            return pltpu.async_copy(x_hbm.at[tv.at[sl]], rows_v.at[sl], sgs[sl])

        g_prev = issue(0, 0)
        scat = [None, None]
        for i in range(nchunks):
            sl = i % 2
            nsl = (i + 1) % 2
            if i + 1 < nchunks:
                if scat[nsl] is not None:
                    scat[nsl].wait()
                    scat[nsl] = None
                g_next = issue(i + 1, nsl)
            g_prev.wait()
            scat[sl] = pltpu.async_copy(
                rows_v.at[sl], out_hbm.at[dv.at[sl]], sss[sl])
            if i + 1 < nchunks:
                g_prev = g_next
        for c in scat:
            if c is not None:
                c.wait()

    return k(x, dest, tok)


def _sc_combine(ys, pos0, pos1, w0x, w1x):
    g_rows, d = ys.shape
    n = pos0.shape[0]
    mesh = plsc.VectorSubcoreMesh(core_axis_name="c", subcore_axis_name="s")
    info = plsc.get_sparse_core_info()
    nw = info.num_cores * info.num_subcores
    per_w = n // nw
    chunk = 8
    nchunks = per_w // chunk

    @functools.partial(
        pl.kernel,
        mesh=mesh,
        out_type=jax.ShapeDtypeStruct((n, d), jnp.float32),
        scratch_types=[
            pltpu.VMEM((2, chunk), jnp.int32),
            pltpu.VMEM((2, chunk), jnp.int32),
            pltpu.VMEM((2, chunk, 16), jnp.float32),
            pltpu.VMEM((2, chunk, 16), jnp.float32),
            pltpu.VMEM((2, chunk, d), jnp.float32),
            pltpu.VMEM((2, chunk, d), jnp.float32),
            pltpu.SemaphoreType.DMA,
            pltpu.SemaphoreType.DMA,
            pltpu.SemaphoreType.DMA,
            pltpu.SemaphoreType.DMA,
            pltpu.SemaphoreType.DMA,
            pltpu.SemaphoreType.DMA,
        ],
    )
    def k(ys_hbm, p0_hbm, p1_hbm, w0_hbm, w1_hbm, out_hbm,
          ia, ib, wa, wb, ra, rb, sa0, sa1, sb0, sb1, so0, so1):
        wid = lax.axis_index("s") * info.num_cores + lax.axis_index("c")
        base = wid * per_w
        sas = (sa0, sa1)
        sbs = (sb0, sb1)
        sos = (so0, so1)
        unroll = 4

        def issue(i, sl):
            off = base + i * chunk
            pltpu.sync_copy(p0_hbm.at[pl.ds(off, chunk)], ia.at[sl])
            pltpu.sync_copy(p1_hbm.at[pl.ds(off, chunk)], ib.at[sl])
            pltpu.sync_copy(w0_hbm.at[pl.ds(off, chunk)], wa.at[sl])
            pltpu.sync_copy(w1_hbm.at[pl.ds(off, chunk)], wb.at[sl])
            ca = pltpu.async_copy(ys_hbm.at[ia.at[sl]], ra.at[sl], sas[sl])
            cb = pltpu.async_copy(ys_hbm.at[ib.at[sl]], rb.at[sl], sbs[sl])
            return (ca, cb)

        pend = issue(0, 0)
        st = [None, None]
        for i in range(nchunks):
            sl = i % 2
            nsl = (i + 1) % 2
            if i + 1 < nchunks:
                if st[nsl] is not None:
                    st[nsl].wait()
                    st[nsl] = None
                nxt = issue(i + 1, nsl)
            pend[0].wait()
            pend[1].wait()
            if st[sl] is not None:
                st[sl].wait()
                st[sl] = None

            def row(r, _):
                bwa = wa[sl, r]
                bwb = wb[sl, r]

                def colstep(c, _):
                    b0 = c * (16 * unroll)
                    for u in range(unroll):
                        cs = pl.ds(b0 + u * 16, 16)
                        ra[sl, r, cs] = ra[sl, r, cs] * bwa + rb[sl, r, cs] * bwb
                    return ()

                lax.fori_loop(0, d // (16 * unroll), colstep, ())
                return ()

            lax.fori_loop(0, chunk, row, ())
            st[sl] = pltpu.async_copy(
                ra.at[sl], out_hbm.at[pl.ds(base + i * chunk, chunk)], sos[sl])
            if i + 1 < nchunks:
                pend = nxt
        for c in st:
            if c is not None:
                c.wait()

    return k(ys, pos0, pos1, w0x, w1x)


def _ffn_body(be_ref, xs_ref, w1_ref, b1_ref, w2_ref, b2_ref,
              out_ref, acc_ref):
    s = pl.program_id(1)
    xb = xs_ref[...].astype(jnp.bfloat16)
    hb = jnp.dot(xb, w1_ref[0].astype(jnp.bfloat16),
                 preferred_element_type=jnp.float32)
    hb = jnp.maximum(hb + b1_ref[0], 0.0)
    contrib = jnp.dot(hb.astype(jnp.bfloat16),
                      w2_ref[0].astype(jnp.bfloat16),
                      preferred_element_type=jnp.float32)

    @pl.when(s == 0)
    def _():
        acc_ref[...] = contrib

    @pl.when(s > 0)
    def _():
        acc_ref[...] = acc_ref[...] + contrib

    @pl.when(s == pl.num_programs(1) - 1)
    def _():
        out_ref[...] = acc_ref[...] + b2_ref[0]


def _ffn(xs, w1r, b1r, w2r, b2r, block_expert, nblocks):
    g_rows, d = xs.shape
    e_num, _, f = w1r.shape
    nj = f // FT

    def jmap(b, s):
        return jnp.where(b % 2 == 0, s, (nj - 1) - s)

    grid_spec = pltpu.PrefetchScalarGridSpec(
        num_scalar_prefetch=1,
        grid=(nblocks, nj),
        in_specs=[
            pl.BlockSpec((BM, d), lambda b, s, be: (b, 0)),
            pl.BlockSpec((1, d, FT), lambda b, s, be: (be[b], 0, jmap(b, s))),
            pl.BlockSpec((1, 1, FT), lambda b, s, be: (be[b], 0, jmap(b, s))),
            pl.BlockSpec((1, FT, d), lambda b, s, be: (be[b], jmap(b, s), 0)),
            pl.BlockSpec((1, 1, d), lambda b, s, be: (be[b], 0, 0)),
        ],
        out_specs=pl.BlockSpec((BM, d), lambda b, s, be: (b, 0)),
        scratch_shapes=[
            pltpu.VMEM((BM, d), jnp.float32),
        ],
    )
    return pl.pallas_call(
        _ffn_body,
        grid_spec=grid_spec,
        out_shape=jax.ShapeDtypeStruct((g_rows, d), jnp.float32),
        compiler_params=pltpu.CompilerParams(
            dimension_semantics=("arbitrary", "arbitrary"),
        ),
    )(block_expert, xs, w1r, b1r, w2r, b2r)


def kernel(inputs, Wg, W1, b1, W2, b2):
    n, d = inputs.shape
    e_num = Wg.shape[1]
    f = W1.shape[2]
    p = n * KTOP

    wg_pad = jnp.pad(Wg, ((0, 0), (0, LANES - e_num)))
    top_idx, top_w = _gate(inputs, wg_pad)
    e_flat = top_idx[:, :KTOP].reshape(-1)

    onehot = (e_flat[:, None] == jnp.arange(e_num)[None, :]).astype(jnp.int32)
    counts = jnp.sum(onehot, axis=0)
    rank = jnp.sum((jnp.cumsum(onehot, axis=0) - onehot) * onehot, axis=1)
    pc = ((counts + BM - 1) // BM) * BM
    cum_pc = jnp.cumsum(pc)
    off = cum_pc - pc
    dest = (off[e_flat] + rank).astype(jnp.int32)

    g_rows = p + e_num * BM
    nblocks = g_rows // BM
    block_start = jnp.arange(nblocks, dtype=jnp.int32) * BM
    block_expert = jnp.minimum(
        jnp.sum((block_start[:, None] >= cum_pc[None, :]).astype(jnp.int32), axis=1),
        e_num - 1).astype(jnp.int32)
    pos = dest.reshape(n, KTOP)

    tok = (jnp.arange(p, dtype=jnp.int32) // KTOP).astype(jnp.int32)
    xs = _sc_dispatch(inputs, dest, tok, g_rows)

    ys = _ffn(xs, W1, b1.reshape(e_num, 1, f), W2, b2.reshape(e_num, 1, d),
              block_expert, nblocks)

    w0x = jnp.broadcast_to(top_w[:, 0:1], (n, 16))
    w1x = jnp.broadcast_to(top_w[:, 1:2], (n, 16))
    return _sc_combine(ys, pos[:, 0], pos[:, 1], w0x, w1x)
